# baseline (speedup 1.0000x reference)
import jax
import jax.numpy as jnp
from jax import lax
from jax.experimental import pallas as pl
from jax.experimental.pallas import tpu as pltpu
from jax.experimental.pallas import tpu_sc as plsc

N = 10000
E = 320000
D = 128
R = 16
NC, NS, L = 2, 16, 16
DH = D // NC
CH = 64
CPB = 16
NB = 20
EPW = NB * CPB * CH
EPAD = NS * EPW
NP = 10240
NPS = NP // NS


def _mm_body(x_ref, w_ref, o_ref):
    o_ref[0] = jnp.dot(x_ref[...], w_ref[0], preferred_element_type=jnp.float32)


def _rel_matmul(x, W):
    BN = 2000
    return pl.pallas_call(
        _mm_body,
        grid=(R, N // BN),
        in_specs=[
            pl.BlockSpec((BN, D), lambda r, n: (n, 0)),
            pl.BlockSpec((1, D, D), lambda r, n: (r, 0, 0)),
        ],
        out_specs=pl.BlockSpec((1, BN, D), lambda r, n: (r, n, 0)),
        out_shape=jax.ShapeDtypeStruct((R, N, D), jnp.float32),
    )(x, W)


def _relu_mm_body(p_ref, b_ref, w_ref, o_ref):
    h = jnp.concatenate([p_ref[0], p_ref[1]], axis=1) + b_ref[0]
    h = jnp.maximum(h, 0.0)
    o_ref[0] = jnp.dot(h, w_ref[0], preferred_element_type=jnp.float32)


def _relu_rel_matmul(p, b, W):
    BN = 2000
    return pl.pallas_call(
        _relu_mm_body,
        grid=(R, N // BN),
        in_specs=[
            pl.BlockSpec((NC, BN, DH), lambda r, n: (0, n, 0)),
            pl.BlockSpec((1, D), lambda r, n: (0, 0)),
            pl.BlockSpec((1, D, D), lambda r, n: (r, 0, 0)),
        ],
        out_specs=pl.BlockSpec((1, BN, D), lambda r, n: (r, n, 0)),
        out_shape=jax.ShapeDtypeStruct((R, N, D), jnp.float32),
    )(p, b.reshape(1, D), W)


def _softmax_body(p_ref, b_ref, o_ref):
    z = jnp.concatenate([p_ref[0], p_ref[1]], axis=1) + b_ref[0]
    z = z - jnp.max(z, axis=1, keepdims=True)
    e = jnp.exp(z)
    o_ref[...] = e / jnp.sum(e, axis=1, keepdims=True)


def _bias_softmax(p, b):
    BN = 2000
    return pl.pallas_call(
        _softmax_body,
        grid=(N // BN,),
        in_specs=[
            pl.BlockSpec((NC, BN, DH), lambda n: (0, n, 0)),
            pl.BlockSpec((1, D), lambda n: (0, 0)),
        ],
        out_specs=pl.BlockSpec((BN, D), lambda n: (n, 0)),
        out_shape=jax.ShapeDtypeStruct((N, D), jnp.float32),
    )(p, b.reshape(1, D))


def _lane_splat(v, j):
    idx = jnp.full((L, 1), j, jnp.int32)
    dnums = lax.GatherDimensionNumbers(
        offset_dims=(), collapsed_slice_dims=(0,), start_index_map=(0,))
    return lax.gather(v, idx, dnums, (1,),
                      mode=lax.GatherScatterMode.PROMISE_IN_BOUNDS)


def _sc_scatter_layer(H, srcs, ets, dsts, norms, feats=None):
    use_feats = feats is not None
    mesh = plsc.VectorSubcoreMesh(core_axis_name="c", subcore_axis_name="s",
                                  num_cores=NC, num_subcores=NS)
    scratch = [
        pltpu.VMEM((CPB, CH), jnp.int32),
        pltpu.VMEM((CPB, CH), jnp.int32),
        pltpu.VMEM((CPB, CH), jnp.int32),
        pltpu.VMEM((CPB, CH), jnp.float32),
        pltpu.VMEM((CH,), jnp.int32),
        pltpu.VMEM((CH, D), jnp.float32),
        pltpu.VMEM((CH, DH), jnp.float32),
        pltpu.VMEM_SHARED((NP, DH), jnp.float32),
        pltpu.SemaphoreType.DMA,
    ]
    if use_feats:
        scratch.append(pltpu.VMEM((CH,), jnp.int32))

    def body(*refs):
        if use_feats:
            (h_hbm, feats_hbm, srcs_hbm, ets_hbm, dsts_hbm, norms_hbm,
             out_hbm, src_v, et_v, dst_v, norm_v, gidx_v, rows_v,
             half_v, acc_sh, sem, fsrc_v) = refs
        else:
            feats_hbm = fsrc_v = None
            (h_hbm, srcs_hbm, ets_hbm, dsts_hbm, norms_hbm,
             out_hbm, src_v, et_v, dst_v, norm_v, gidx_v, rows_v,
             half_v, acc_sh, sem) = refs
        cid = lax.axis_index("c")
        sid = lax.axis_index("s")
        hoff = cid * DH

        zeros16 = jnp.zeros((L,), jnp.float32)

        def zero_rows(i, _):
            for k in range(DH // L):
                half_v[i, pl.ds(k * L, L)] = zeros16
            return 0
        lax.fori_loop(0, CH, zero_rows, 0)
        for j in range(NPS // CH):
            pltpu.sync_copy(half_v,
                            acc_sh.at[pl.ds(sid * NPS + j * CH, CH)])
        plsc.subcore_barrier()

        def block(b, _):
            pltpu.sync_copy(srcs_hbm.at[sid, b], src_v)
            pltpu.sync_copy(ets_hbm.at[sid, b], et_v)
            pltpu.sync_copy(dsts_hbm.at[sid, b], dst_v)
            pltpu.sync_copy(norms_hbm.at[sid, b], norm_v)

            def chunk(c, _):
                if use_feats:
                    pltpu.async_copy(feats_hbm.at[src_v.at[c]],
                                     fsrc_v, sem).wait()
                    for k in range(CH // L):
                        gidx_v[pl.ds(k * L, L)] = (
                            et_v[c, pl.ds(k * L, L)] * N
                            + fsrc_v[pl.ds(k * L, L)])
                else:
                    for k in range(CH // L):
                        gidx_v[pl.ds(k * L, L)] = (
                            et_v[c, pl.ds(k * L, L)] * N
                            + src_v[c, pl.ds(k * L, L)])
                pltpu.async_copy(h_hbm.at[gidx_v], rows_v, sem).wait()

                def scale(h, _):
                    n16 = norm_v[c, pl.ds(h * L, L)]
                    for j in range(L):
                        s = _lane_splat(n16, j)
                        row = h * L + j
                        for k in range(DH // L):
                            half_v[row, pl.ds(k * L, L)] = (
                                rows_v[row, pl.ds(hoff + k * L, L)] * s)
                    return 0
                lax.fori_loop(0, CH // L, scale, 0)
                pltpu.sync_copy(half_v, acc_sh.at[dst_v.at[c]], add=True)
                return 0
            lax.fori_loop(0, CPB, chunk, 0)
            return 0
        lax.fori_loop(0, NB, block, 0)

        plsc.subcore_barrier()
        pltpu.sync_copy(acc_sh.at[pl.ds(sid * NPS, NPS)],
                        out_hbm.at[cid, pl.ds(sid * NPS, NPS)])

    kern = pl.kernel(body,
                     out_type=jax.ShapeDtypeStruct((NC, NP, DH), jnp.float32),
                     mesh=mesh, scratch_types=scratch)
    if use_feats:
        return kern(H, feats, srcs, ets, dsts, norms)
    return kern(H, srcs, ets, dsts, norms)


def kernel(feats, edge_index, edge_type, edge_norm, emb, W1, b1, W2, b2):
    feats = feats.astype(jnp.int32)
    src = edge_index[0].astype(jnp.int32)
    dst = edge_index[1].astype(jnp.int32)
    et = edge_type.astype(jnp.int32)
    nrm = edge_norm[:, 0].astype(jnp.float32)

    pad = EPAD - E

    def shape_edges(a):
        return jnp.concatenate(
            [a, jnp.zeros((pad,), a.dtype)]).reshape(NS, NB, CPB, CH)

    srcs = shape_edges(src)
    ets = shape_edges(et)
    dsts = shape_edges(dst)
    norms = shape_edges(nrm)

    H1 = _rel_matmul(emb, W1).reshape(R * N, D)
    p1 = _sc_scatter_layer(H1, srcs, ets, dsts, norms, feats=feats)
    H2 = _relu_rel_matmul(p1, b1, W2).reshape(R * N, D)
    p2 = _sc_scatter_layer(H2, srcs, ets, dsts, norms)
    return _bias_softmax(p2, b2)


# speedup vs baseline: 1.1941x; 1.1941x over previous
import jax
import jax.numpy as jnp
from jax import lax
from jax.experimental import pallas as pl
from jax.experimental.pallas import tpu as pltpu
from jax.experimental.pallas import tpu_sc as plsc

N = 10000
E = 320000
D = 128
R = 16
NC, NS, L = 2, 16, 16
DH = D // NC
CH = 64
CPB = 10
NB = 32
EPW = NB * CPB * CH
EPAD = NS * EPW
NP = 10240
NPS = NP // NS


def _mm_body(x_ref, w_ref, o_ref):
    o_ref[0] = jnp.dot(x_ref[...], w_ref[0], preferred_element_type=jnp.float32)


def _rel_matmul(x, W):
    BN = 2000
    return pl.pallas_call(
        _mm_body,
        grid=(R, N // BN),
        in_specs=[
            pl.BlockSpec((BN, D), lambda r, n: (n, 0)),
            pl.BlockSpec((1, D, D), lambda r, n: (r, 0, 0)),
        ],
        out_specs=pl.BlockSpec((1, BN, D), lambda r, n: (r, n, 0)),
        out_shape=jax.ShapeDtypeStruct((R, N, D), jnp.float32),
    )(x, W)


def _relu_mm_body(p_ref, b_ref, w_ref, o_ref):
    h = jnp.concatenate([p_ref[0], p_ref[1]], axis=1) + b_ref[0]
    h = jnp.maximum(h, 0.0)
    o_ref[0] = jnp.dot(h, w_ref[0], preferred_element_type=jnp.float32)


def _relu_rel_matmul(p, b, W):
    BN = 2000
    return pl.pallas_call(
        _relu_mm_body,
        grid=(R, N // BN),
        in_specs=[
            pl.BlockSpec((NC, BN, DH), lambda r, n: (0, n, 0)),
            pl.BlockSpec((1, D), lambda r, n: (0, 0)),
            pl.BlockSpec((1, D, D), lambda r, n: (r, 0, 0)),
        ],
        out_specs=pl.BlockSpec((1, BN, D), lambda r, n: (r, n, 0)),
        out_shape=jax.ShapeDtypeStruct((R, N, D), jnp.float32),
    )(p, b.reshape(1, D), W)


def _softmax_body(p_ref, b_ref, o_ref):
    z = jnp.concatenate([p_ref[0], p_ref[1]], axis=1) + b_ref[0]
    z = z - jnp.max(z, axis=1, keepdims=True)
    e = jnp.exp(z)
    o_ref[...] = e / jnp.sum(e, axis=1, keepdims=True)


def _bias_softmax(p, b):
    BN = 2000
    return pl.pallas_call(
        _softmax_body,
        grid=(N // BN,),
        in_specs=[
            pl.BlockSpec((NC, BN, DH), lambda n: (0, n, 0)),
            pl.BlockSpec((1, D), lambda n: (0, 0)),
        ],
        out_specs=pl.BlockSpec((BN, D), lambda n: (n, 0)),
        out_shape=jax.ShapeDtypeStruct((N, D), jnp.float32),
    )(p, b.reshape(1, D))


def _lane_splat(v, j):
    idx = jnp.full((L, 1), j, jnp.int32)
    dnums = lax.GatherDimensionNumbers(
        offset_dims=(), collapsed_slice_dims=(0,), start_index_map=(0,))
    return lax.gather(v, idx, dnums, (1,),
                      mode=lax.GatherScatterMode.PROMISE_IN_BOUNDS)


def _sc_scatter_layer(H, srcs, ets, dsts, norms, feats=None):
    use_feats = feats is not None
    mesh = plsc.VectorSubcoreMesh(core_axis_name="c", subcore_axis_name="s",
                                  num_cores=NC, num_subcores=NS)
    scratch = [
        pltpu.VMEM((CPB, CH), jnp.int32),
        pltpu.VMEM((CPB, CH), jnp.int32),
        pltpu.VMEM((CPB, CH), jnp.int32),
        pltpu.VMEM((CPB, CH), jnp.float32),
        pltpu.VMEM((CPB, CH), jnp.int32),
        pltpu.VMEM((CH, D), jnp.float32),
        pltpu.VMEM((CH, D), jnp.float32),
        pltpu.VMEM((CH, DH), jnp.float32),
        pltpu.VMEM_SHARED((NP, DH), jnp.float32),
        pltpu.SemaphoreType.DMA,
        pltpu.SemaphoreType.DMA,
    ]
    if use_feats:
        scratch.append(pltpu.VMEM((CPB, CH), jnp.int32))
        scratch.append(pltpu.SemaphoreType.DMA)

    def body(*refs):
        if use_feats:
            (h_hbm, feats_hbm, srcs_hbm, ets_hbm, dsts_hbm, norms_hbm,
             out_hbm, src_v, et_v, dst_v, norm_v, gidx_v, rows_a, rows_b,
             half_v, acc_sh, sem_a, sem_b, fsrc_v, fsem) = refs
        else:
            feats_hbm = fsrc_v = fsem = None
            (h_hbm, srcs_hbm, ets_hbm, dsts_hbm, norms_hbm,
             out_hbm, src_v, et_v, dst_v, norm_v, gidx_v, rows_a, rows_b,
             half_v, acc_sh, sem_a, sem_b) = refs
        cid = lax.axis_index("c")
        sid = lax.axis_index("s")
        hoff = cid * DH

        zeros16 = jnp.zeros((L,), jnp.float32)

        def zero_rows(i, _):
            for k in range(DH // L):
                half_v[i, pl.ds(k * L, L)] = zeros16
            return 0
        lax.fori_loop(0, CH, zero_rows, 0)
        for j in range(NPS // CH):
            pltpu.sync_copy(half_v,
                            acc_sh.at[pl.ds(sid * NPS + j * CH, CH)])
        plsc.subcore_barrier()

        def fire(c, rows_v, sem):
            pltpu.async_copy(h_hbm.at[gidx_v.at[c]], rows_v, sem)

        def drain(c, rows_v, sem):
            pltpu.make_async_copy(h_hbm.at[gidx_v.at[c]], rows_v, sem).wait()

        def scale_scatter(c, rows_v):
            def scale(h, _):
                n16 = norm_v[c, pl.ds(h * L, L)]
                for j in range(L):
                    s = _lane_splat(n16, j)
                    row = h * L + j
                    for k in range(DH // L):
                        half_v[row, pl.ds(k * L, L)] = (
                            rows_v[row, pl.ds(hoff + k * L, L)] * s)
                return 0
            lax.fori_loop(0, CH // L, scale, 0)
            pltpu.sync_copy(half_v, acc_sh.at[dst_v.at[c]], add=True)

        def block(b, _):
            pltpu.sync_copy(srcs_hbm.at[sid, b], src_v)
            pltpu.sync_copy(ets_hbm.at[sid, b], et_v)
            pltpu.sync_copy(dsts_hbm.at[sid, b], dst_v)
            pltpu.sync_copy(norms_hbm.at[sid, b], norm_v)

            if use_feats:
                descs = [pltpu.async_copy(feats_hbm.at[src_v.at[c]],
                                          fsrc_v.at[c], fsem)
                         for c in range(CPB)]
                for d in descs:
                    d.wait()

            def bgidx(c, _):
                for k in range(CH // L):
                    s16 = (fsrc_v if use_feats else src_v)[c, pl.ds(k * L, L)]
                    gidx_v[c, pl.ds(k * L, L)] = (
                        et_v[c, pl.ds(k * L, L)] * N + s16)
                return 0
            lax.fori_loop(0, CPB, bgidx, 0)

            fire(0, rows_a, sem_a)

            def pair(p, _):
                c0 = 2 * p
                c1 = c0 + 1
                fire(c1, rows_b, sem_b)
                drain(c0, rows_a, sem_a)
                scale_scatter(c0, rows_a)

                @pl.when(c1 + 1 < CPB)
                def _():
                    fire(c1 + 1, rows_a, sem_a)
                drain(c1, rows_b, sem_b)
                scale_scatter(c1, rows_b)
                return 0
            lax.fori_loop(0, CPB // 2, pair, 0)
            return 0
        lax.fori_loop(0, NB, block, 0)

        plsc.subcore_barrier()
        pltpu.sync_copy(acc_sh.at[pl.ds(sid * NPS, NPS)],
                        out_hbm.at[cid, pl.ds(sid * NPS, NPS)])

    kern = pl.kernel(body,
                     out_type=jax.ShapeDtypeStruct((NC, NP, DH), jnp.float32),
                     mesh=mesh, scratch_types=scratch)
    if use_feats:
        return kern(H, feats, srcs, ets, dsts, norms)
    return kern(H, srcs, ets, dsts, norms)


def kernel(feats, edge_index, edge_type, edge_norm, emb, W1, b1, W2, b2):
    feats = feats.astype(jnp.int32)
    src = edge_index[0].astype(jnp.int32)
    dst = edge_index[1].astype(jnp.int32)
    et = edge_type.astype(jnp.int32)
    nrm = edge_norm[:, 0].astype(jnp.float32)

    pad = EPAD - E

    def shape_edges(a):
        return jnp.concatenate(
            [a, jnp.zeros((pad,), a.dtype)]).reshape(NS, NB, CPB, CH)

    srcs = shape_edges(src)
    ets = shape_edges(et)
    dsts = shape_edges(dst)
    norms = shape_edges(nrm)

    H1 = _rel_matmul(emb, W1).reshape(R * N, D)
    p1 = _sc_scatter_layer(H1, srcs, ets, dsts, norms, feats=feats)
    H2 = _relu_rel_matmul(p1, b1, W2).reshape(R * N, D)
    p2 = _sc_scatter_layer(H2, srcs, ets, dsts, norms)
    return _bias_softmax(p2, b2)
